# baseline (speedup 1.0000x reference)
import functools

import jax
import jax.numpy as jnp
from jax import lax
from jax.experimental import pallas as pl
from jax.experimental.pallas import tpu as pltpu
from jax.experimental.pallas import tpu_sc as plsc

B, L, C = 4096, 200, 128
VOCAB = 262
NUM_CORES = 2
NUM_SUBCORES = 16
NW = NUM_CORES * NUM_SUBCORES
RPW = B // NW
NBUF = 2
NG = RPW // NBUF
G0, G1 = 128, L - 128
PCHUNKS = ((0, 88), (88, 88), (176, 86))


def _sc_body(idx_hbm, table_hbm, tab_ev_hbm, tab_od_hbm, x_hbm, p_hbm,
             pooled_hbm, pa_v, pb_v, pc_v, idx_v, xrows_v, prows_v,
             sem_i, sem_g, sem_o):
    c = lax.axis_index("c")
    s = lax.axis_index("s")
    wid = s * NUM_CORES + c

    @pl.when(s == 0)
    def _():
        for off, nrows in PCHUNKS:
            rows = pl.ds(off, nrows)
            pltpu.sync_copy(tab_ev_hbm.at[rows], pa_v.at[pl.ds(0, nrows)])
            pltpu.sync_copy(tab_od_hbm.at[rows], pb_v.at[pl.ds(0, nrows)])

            def row_body(r, carry):
                for j in range(4):
                    sl = pl.ds(j * 16, 16)
                    pc_v[r, sl] = jnp.maximum(pa_v[r, sl], pb_v[r, sl])
                return carry

            lax.fori_loop(0, nrows, row_body, 0)
            pltpu.sync_copy(pc_v.at[pl.ds(0, nrows)], pooled_hbm.at[rows])

    plsc.subcore_barrier()

    base_w = wid * RPW

    def idx_copy(i, k):
        return pltpu.make_async_copy(
            idx_hbm.at[base_w + i], idx_v.at[k], sem_i.at[k])

    def g_copies(k):
        return (
            pltpu.make_async_copy(
                table_hbm.at[idx_v.at[k, pl.ds(0, G0)]],
                xrows_v.at[k, pl.ds(0, G0)], sem_g.at[k]),
            pltpu.make_async_copy(
                table_hbm.at[idx_v.at[k, pl.ds(G0, G1)]],
                xrows_v.at[k, pl.ds(G0, G1)], sem_g.at[k]),
            pltpu.make_async_copy(
                pooled_hbm.at[idx_v.at[k, pl.ds(0, G0)]],
                prows_v.at[k, pl.ds(0, G0)], sem_g.at[k]),
            pltpu.make_async_copy(
                pooled_hbm.at[idx_v.at[k, pl.ds(G0, G1)]],
                prows_v.at[k, pl.ds(G0, G1)], sem_g.at[k]),
        )

    def o_copies(i, k):
        return (
            pltpu.make_async_copy(
                xrows_v.at[k], x_hbm.at[base_w + i], sem_o.at[k]),
            pltpu.make_async_copy(
                prows_v.at[k], p_hbm.at[base_w + i], sem_o.at[k]),
        )

    def start_all(copies):
        for cp in copies:
            cp.start()

    def wait_all(copies):
        for cp in copies:
            cp.wait()

    for k in range(NBUF):
        idx_copy(k, k).start()
    for k in range(NBUF):
        idx_copy(k, k).wait()
        start_all(g_copies(k))

    def ring_body(g, carry):
        for k in range(NBUF):
            i = g * NBUF + k
            wait_all(g_copies(k))
            start_all(o_copies(i, k))

            @pl.when(g < NG - 1)
            def _():
                j = i + NBUF
                idx_copy(j, k).start()
                wait_all(o_copies(i, k))
                idx_copy(j, k).wait()
                start_all(g_copies(k))
        return carry

    lax.fori_loop(0, NG, ring_body, 0)

    for k in range(NBUF):
        i = (NG - 1) * NBUF + k
        wait_all(o_copies(i, k))


@jax.jit
def kernel(inputs, table):
    idx = inputs.astype(jnp.int32)
    table = table.astype(jnp.float32)
    tab_ev = table[:, 0::2]
    tab_od = table[:, 1::2]
    mesh = plsc.VectorSubcoreMesh(core_axis_name="c", subcore_axis_name="s")
    call = pl.kernel(
        _sc_body,
        mesh=mesh,
        compiler_params=pltpu.CompilerParams(use_tc_tiling_on_sc=False),
        out_type=[
            jax.ShapeDtypeStruct((B, L, C), jnp.float32),
            jax.ShapeDtypeStruct((B, L, C // 2), jnp.float32),
            jax.ShapeDtypeStruct((VOCAB, C // 2), jnp.float32),
        ],
        scratch_types=[
            pltpu.VMEM((88, C // 2), jnp.float32),
            pltpu.VMEM((88, C // 2), jnp.float32),
            pltpu.VMEM((88, C // 2), jnp.float32),
            pltpu.VMEM((NBUF, L), jnp.int32),
            pltpu.VMEM((NBUF, L, C), jnp.float32),
            pltpu.VMEM((NBUF, L, C // 2), jnp.float32),
            pltpu.SemaphoreType.DMA((NBUF,)),
            pltpu.SemaphoreType.DMA((NBUF,)),
            pltpu.SemaphoreType.DMA((NBUF,)),
        ],
    )
    x, p, _pooled = call(idx, table, tab_ev, tab_od)
    return (x, p)
